# baseline (speedup 1.0000x reference)
import functools

import jax
import jax.numpy as jnp
from jax import lax
from jax.experimental import pallas as pl
from jax.experimental.pallas import tpu as pltpu
from jax.experimental.pallas import tpu_sc as plsc

_DIM = 32
_NW = 32
_PACK = 128 // _DIM
_HISTP = 256
_NBUF = 2
_SPLITS = ((0, 128), (128, 72))


def _make_gather(n_lookups, hist):
    n_chunks = n_lookups // (_PACK * hist)
    chunks_per_w = n_chunks // _NW
    out_rows = n_lookups * _DIM // 128
    assert chunks_per_w % _NBUF == 0
    mesh = plsc.VectorSubcoreMesh(core_axis_name="c", subcore_axis_name="s")

    @functools.partial(
        pl.kernel,
        out_type=jax.ShapeDtypeStruct((out_rows, 128), jnp.float32),
        mesh=mesh,
        scratch_types=[
            pltpu.VMEM((_NBUF, _PACK, _HISTP), jnp.int32),
            pltpu.VMEM((_NBUF, _PACK, hist, _DIM), jnp.float32),
            pltpu.SemaphoreType.DMA((_NBUF,)),
            pltpu.SemaphoreType.DMA((_NBUF,)),
            pltpu.SemaphoreType.DMA((_NBUF,)),
        ],
        compiler_params=pltpu.CompilerParams(use_tc_tiling_on_sc=False),
    )
    def gather_kernel(idx_hbm, table_hbm, out_hbm, idx_v, rows_v,
                      idx_sem, gat_sem, wb_sem):
        wid = lax.axis_index("s") * 2 + lax.axis_index("c")
        chunk0 = wid * chunks_per_w

        def idx_copy(g, b):
            return pltpu.make_async_copy(
                idx_hbm.at[pl.ds((chunk0 + g) * _PACK, _PACK)],
                idx_v.at[b], idx_sem.at[b])

        def wb_copies(g, b):
            return [
                pltpu.make_async_copy(
                    rows_v.at[b, k],
                    out_hbm.at[pl.ds((chunk0 + g) * hist, hist),
                               pl.ds(k * _DIM, _DIM)],
                    wb_sem.at[b])
                for k in range(_PACK)
            ]

        idx_copy(0, 0).start()

        def body(gg, carry):
            for b in range(_NBUF):
                g = gg * _NBUF + b
                idx_copy(g, b).wait()

                @pl.when(g + 1 < chunks_per_w)
                def _():
                    idx_copy(g + 1, (b + 1) % _NBUF).start()

                @pl.when(g >= _NBUF)
                def _():
                    for c in wb_copies(g - _NBUF, b):
                        c.wait()

                copies = [
                    pltpu.async_copy(
                        table_hbm.at[idx_v.at[b, k, pl.ds(lo, ln)]],
                        rows_v.at[b, k, pl.ds(lo, ln)],
                        gat_sem.at[b],
                    )
                    for k in range(_PACK)
                    for (lo, ln) in _SPLITS
                ]
                for c in copies:
                    c.wait()
                for c in wb_copies(g, b):
                    c.start()
            return carry

        lax.fori_loop(0, chunks_per_w // _NBUF, body, 0)
        for b in range(_NBUF):
            for c in wb_copies(chunks_per_w - _NBUF + b, b):
                c.wait()

    return gather_kernel


def kernel(indices, table):
    b, h = indices.shape
    n = b * h
    idx_r = (indices.reshape(n // (_PACK * h), h, _PACK)
             .transpose(0, 2, 1)
             .reshape(n // h, h))
    idx_p = jnp.pad(idx_r, ((0, 0), (0, _HISTP - h)))
    out = _make_gather(n, h)(idx_p, table)
    return out.reshape(b, h, _DIM)
